# baseline (speedup 1.0000x reference)
import functools

import jax
import jax.numpy as jnp
from jax import lax
from jax.experimental import pallas as pl
from jax.experimental.pallas import tpu as pltpu
from jax.experimental.pallas import tpu_sc as plsc

_S = 8192
_D = 1024

_R_SC = 2560
_R_TC = _S - _R_SC
_NS = 16
_R_HALF = _R_SC // 2
_ROWS_W = _R_HALF // _NS
_C = 16
_NCHUNK = _ROWS_W // _C
_NBUF = 3

_mesh1 = plsc.VectorSubcoreMesh(
    core_axis_name="c", subcore_axis_name="s", num_cores=1)

_sc_scratch = (
    [pltpu.VMEM((_NCHUNK, _C), jnp.int32)]
    + [pltpu.VMEM((_C, _D), jnp.float32)] * (2 * _NBUF)
    + [pltpu.SemaphoreType.DMA] * (2 * _NBUF)
)


def _posenc_sc_half(x_hbm, enc_hbm, out_hbm, idx_v, *bufs_and_sems, half):
    bx = bufs_and_sems[0:_NBUF]
    be = bufs_and_sems[_NBUF:2 * _NBUF]
    sem_l = bufs_and_sems[2 * _NBUF:3 * _NBUF]
    sem_s = bufs_and_sems[3 * _NBUF:4 * _NBUF]

    wid = lax.axis_index("s")
    obase = wid * _ROWS_W
    wbase = _R_TC + half * _R_HALF + obase
    iota16 = lax.iota(jnp.int32, 16)
    for k in range(_NCHUNK):
        for j in range(_C // 16):
            idx_v[k, pl.ds(j * 16, 16)] = wbase + (k * _C + j * 16) + iota16

    def x_copy(k):
        b = k % _NBUF
        return pltpu.make_async_copy(
            x_hbm.at[pl.ds(wbase + k * _C, _C)], bx[b], sem_l[b])

    def e_copy(k):
        b = k % _NBUF
        return pltpu.make_async_copy(enc_hbm.at[idx_v.at[k]], be[b], sem_l[b])

    def s_copy(k):
        b = k % _NBUF
        return pltpu.make_async_copy(
            bx[b], out_hbm.at[pl.ds(obase + k * _C, _C)], sem_s[b])

    def start_load(k):
        x_copy(k).start()
        e_copy(k).start()

    start_load(0)
    if _NCHUNK > 1:
        start_load(1)
    for k in range(_NCHUNK):
        x_copy(k).wait()
        e_copy(k).wait()
        b = k % _NBUF

        @pl.loop(0, _C)
        def _add_row(r, _b=b):
            for j in range(_D // 16):
                s = pl.ds(j * 16, 16)
                bx[_b][r, s] = bx[_b][r, s] + be[_b][r, s]

        s_copy(k).start()
        if k + 2 < _NCHUNK:
            if k >= 1:
                s_copy(k - 1).wait()
            start_load(k + 2)
    for k in range(max(0, _NCHUNK - 3), _NCHUNK):
        s_copy(k).wait()


def _make_sc_half(half):
    return functools.partial(
        pl.kernel,
        out_type=jax.ShapeDtypeStruct((_R_HALF, _D), jnp.float32),
        mesh=_mesh1,
        scratch_types=_sc_scratch,
    )(functools.partial(_posenc_sc_half, half=half))


_sc_half0 = _make_sc_half(0)
_sc_half1 = _make_sc_half(1)


_BLOCK_S = 512


def _add_block(x_ref, enc_ref, out_ref):
    out_ref[...] = x_ref[...] + enc_ref[...]


def _posenc_tc(x2, encoding):
    return pl.pallas_call(
        _add_block,
        grid=(_R_TC // _BLOCK_S,),
        in_specs=[
            pl.BlockSpec((_BLOCK_S, _D), lambda i: (i, 0)),
            pl.BlockSpec((_BLOCK_S, _D), lambda i: (i, 0)),
        ],
        out_specs=pl.BlockSpec((_BLOCK_S, _D), lambda i: (i, 0)),
        out_shape=jax.ShapeDtypeStruct((_S, _D), jnp.float32),
    )(x2, encoding)


def kernel(x, encoding):
    N, S, D = x.shape
    x2 = x.reshape(S, D)
    tc_out = _posenc_tc(x2, encoding)
    sc0 = _sc_half0(x2, encoding)
    sc1 = _sc_half1(x2, encoding)
    out = lax.dynamic_update_slice(tc_out, sc0, (_R_TC, 0))
    out = lax.dynamic_update_slice(out, sc1, (_R_TC + _R_HALF, 0))
    return out.reshape(N, S, D)


# speedup vs baseline: 1.0589x; 1.0589x over previous
import functools

import jax
import jax.numpy as jnp
from jax import lax
from jax.experimental import pallas as pl
from jax.experimental.pallas import tpu as pltpu
from jax.experimental.pallas import tpu_sc as plsc

_S = 8192
_D = 1024

_R_SC = 2560
_R_TC = _S - _R_SC
_NS = 16
_ROWS_W = _R_SC // _NS
_C = 16
_NCHUNK = _ROWS_W // _C
_NBUF = 3

_mesh = plsc.VectorSubcoreMesh(
    core_axis_name="c", subcore_axis_name="s", num_cores=1)


@functools.partial(
    pl.kernel,
    out_type=jax.ShapeDtypeStruct((_R_SC, _D), jnp.float32),
    mesh=_mesh,
    scratch_types=(
        [pltpu.VMEM((_NCHUNK, _C), jnp.int32)]
        + [pltpu.VMEM((_C, _D), jnp.float32)] * (2 * _NBUF)
        + [pltpu.SemaphoreType.DMA] * (2 * _NBUF)
    ),
)
def _posenc_sc(x_hbm, enc_hbm, out_hbm, idx_v, *bufs_and_sems):
    bx = bufs_and_sems[0:_NBUF]
    be = bufs_and_sems[_NBUF:2 * _NBUF]
    sem_l = bufs_and_sems[2 * _NBUF:3 * _NBUF]
    sem_s = bufs_and_sems[3 * _NBUF:4 * _NBUF]

    wid = lax.axis_index("s")
    obase = wid * _ROWS_W
    wbase = _R_TC + obase
    iota16 = lax.iota(jnp.int32, 16)
    for k in range(_NCHUNK):
        for j in range(_C // 16):
            idx_v[k, pl.ds(j * 16, 16)] = wbase + (k * _C + j * 16) + iota16

    def x_copy(k):
        b = k % _NBUF
        return pltpu.make_async_copy(
            x_hbm.at[pl.ds(wbase + k * _C, _C)], bx[b], sem_l[b])

    def e_copy(k):
        b = k % _NBUF
        return pltpu.make_async_copy(enc_hbm.at[idx_v.at[k]], be[b], sem_l[b])

    def s_copy(k):
        b = k % _NBUF
        return pltpu.make_async_copy(
            bx[b], out_hbm.at[pl.ds(obase + k * _C, _C)], sem_s[b])

    def start_load(k):
        x_copy(k).start()
        e_copy(k).start()

    start_load(0)
    start_load(1)
    for k in range(_NCHUNK):
        x_copy(k).wait()
        e_copy(k).wait()
        b = k % _NBUF

        @pl.loop(0, _C)
        def _add_row(r, _b=b):
            for j in range(_D // 16):
                s = pl.ds(j * 16, 16)
                bx[_b][r, s] = bx[_b][r, s] + be[_b][r, s]

        s_copy(k).start()
        if k + 2 < _NCHUNK:
            if k >= 1:
                s_copy(k - 1).wait()
            start_load(k + 2)
    for k in range(_NCHUNK - 3, _NCHUNK):
        s_copy(k).wait()


_BLOCK_S = 512


def _add_block(x_ref, enc_ref, out_ref):
    out_ref[...] = x_ref[...] + enc_ref[...]


def _posenc_tc(x2, encoding):
    return pl.pallas_call(
        _add_block,
        grid=(_R_TC // _BLOCK_S,),
        in_specs=[
            pl.BlockSpec((_BLOCK_S, _D), lambda i: (i, 0)),
            pl.BlockSpec((_BLOCK_S, _D), lambda i: (i, 0)),
        ],
        out_specs=pl.BlockSpec((_BLOCK_S, _D), lambda i: (i, 0)),
        out_shape=jax.ShapeDtypeStruct((_S, _D), jnp.float32),
    )(x2, encoding)


def kernel(x, encoding):
    N, S, D = x.shape
    x2 = x.reshape(S, D)
    tc_out = _posenc_tc(x2, encoding)
    sc_out = _posenc_sc(x2, encoding)
    out = lax.dynamic_update_slice(tc_out, sc_out, (_R_TC, 0))
    return out.reshape(N, S, D)


# speedup vs baseline: 1.3000x; 1.2278x over previous
import functools

import jax
import jax.numpy as jnp
from jax import lax
from jax.experimental import pallas as pl
from jax.experimental.pallas import tpu as pltpu
from jax.experimental.pallas import tpu_sc as plsc

_S = 8192
_D = 1024

_R_SC = 512
_R_TC = _S - _R_SC
_NC = 2
_NS = 16
_NW = _NC * _NS
_C = _R_SC // _NW

_mesh = plsc.VectorSubcoreMesh(core_axis_name="c", subcore_axis_name="s")


@functools.partial(
    pl.kernel,
    out_type=jax.ShapeDtypeStruct((_R_SC, _D), jnp.float32),
    mesh=_mesh,
    scratch_types=[
        pltpu.VMEM((1, _C), jnp.int32),
        pltpu.VMEM((_C, _D), jnp.float32),
        pltpu.VMEM((_C, _D), jnp.float32),
        pltpu.SemaphoreType.DMA,
    ],
)
def _posenc_sc(x_hbm, enc_hbm, out_hbm, idx_v, bufx, bufe, sem):
    wid = lax.axis_index("s") * _NC + lax.axis_index("c")
    obase = wid * _C
    wbase = _R_TC + obase
    idx_v[0, pl.ds(0, 16)] = wbase + lax.iota(jnp.int32, 16)

    cpx = pltpu.make_async_copy(x_hbm.at[pl.ds(wbase, _C)], bufx, sem)
    cpe = pltpu.make_async_copy(enc_hbm.at[idx_v.at[0]], bufe, sem)
    cpx.start()
    cpe.start()
    cpx.wait()
    cpe.wait()

    @pl.loop(0, _C)
    def _add_row(r):
        for j in range(_D // 16):
            s = pl.ds(j * 16, 16)
            bufx[r, s] = bufx[r, s] + bufe[r, s]

    pltpu.sync_copy(bufx, out_hbm.at[pl.ds(obase, _C)])


_BLOCK_S = 512


def _add_block(x_ref, enc_ref, out_ref):
    out_ref[...] = x_ref[...] + enc_ref[...]


def _posenc_tc(x2, encoding):
    return pl.pallas_call(
        _add_block,
        grid=(_R_TC // _BLOCK_S,),
        in_specs=[
            pl.BlockSpec((_BLOCK_S, _D), lambda i: (i, 0)),
            pl.BlockSpec((_BLOCK_S, _D), lambda i: (i, 0)),
        ],
        out_specs=pl.BlockSpec((_BLOCK_S, _D), lambda i: (i, 0)),
        out_shape=jax.ShapeDtypeStruct((_S, _D), jnp.float32),
    )(x2, encoding)


def kernel(x, encoding):
    N, S, D = x.shape
    x2 = x.reshape(S, D)
    tc_out = _posenc_tc(x2, encoding)
    sc_out = _posenc_sc(x2, encoding)
    out = lax.dynamic_update_slice(tc_out, sc_out, (_R_TC, 0))
    return out.reshape(N, S, D)
